# baseline (speedup 1.0000x reference)
import functools
import math

import jax
import jax.numpy as jnp
from jax import lax
from jax.experimental import pallas as pl
from jax.experimental.pallas import tpu as pltpu
from jax.experimental.pallas import tpu_sc as plsc

N = 10000
D = 128
ALPHA = 0.1
LAMBDA = 1.0
LAYER = 4
BETA = math.log(LAMBDA / LAYER + 1.0)

NC = 2
NS = 16
NW = NC * NS
CHUNK = 32
NBUF = 8
DCHUNK = 128
SB = 32
N_PAD = 10240
RPT = N_PAD // NS

_mesh = plsc.VectorSubcoreMesh(core_axis_name="c", subcore_axis_name="s")


def _split(num_edges):
    assert num_edges % CHUNK == 0
    groups = -(-num_edges // (NW * CHUNK))
    groups += (-groups) % SB
    edges_per_worker = groups * CHUNK
    last_real = num_edges - (NW - 1) * edges_per_worker
    assert 0 < last_real <= edges_per_worker and last_real % CHUNK == 0
    assert last_real % NBUF == 0
    last_groups = last_real // CHUNK
    return groups, last_groups


def _make_deg_kernel(groups, last_groups):
    @functools.partial(
        pl.kernel,
        out_type=jax.ShapeDtypeStruct((NC, 2, N_PAD), jnp.float32),
        mesh=_mesh,
        scratch_types=[
            pltpu.VMEM((groups, DCHUNK), jnp.int32),
            pltpu.VMEM((groups, DCHUNK), jnp.int32),
            pltpu.VMEM((DCHUNK,), jnp.float32),
            pltpu.VMEM_SHARED((N_PAD,), jnp.float32),
            pltpu.VMEM_SHARED((N_PAD,), jnp.float32),
            pltpu.SemaphoreType.DMA,
        ],
    )
    def deg_kernel(edges_hbm, ones_hbm, zeros_hbm, out_hbm,
                   src_v, dst_v, ones_v, hsrc_sh, hdst_sh, sem):
        cid = lax.axis_index("c")
        sid = lax.axis_index("s")
        wid = sid * NC + cid
        n_me = jnp.where(wid == NW - 1, last_groups, groups)
        pltpu.sync_copy(edges_hbm.at[0, wid], src_v)
        pltpu.sync_copy(edges_hbm.at[1, wid], dst_v)
        pltpu.sync_copy(ones_hbm, ones_v)
        pltpu.sync_copy(zeros_hbm, hsrc_sh.at[pl.ds(sid * RPT, RPT)])
        pltpu.sync_copy(zeros_hbm, hdst_sh.at[pl.ds(sid * RPT, RPT)])
        plsc.subcore_barrier()

        def body(j, carry):
            d1 = pltpu.async_copy(ones_v, hsrc_sh.at[src_v.at[j]], sem,
                                  add=True)
            d2 = pltpu.async_copy(ones_v, hdst_sh.at[dst_v.at[j]], sem,
                                  add=True)
            d1.wait()
            d2.wait()
            return carry

        lax.fori_loop(0, n_me, body, 0)
        plsc.subcore_barrier()
        sl = pl.ds(sid * RPT, RPT)
        pltpu.sync_copy(hsrc_sh.at[sl], out_hbm.at[cid, 0, sl])
        pltpu.sync_copy(hdst_sh.at[sl], out_hbm.at[cid, 1, sl])

    return deg_kernel


def _make_agg_kernel(groups, last_groups):
    assert groups % SB == 0 and SB % NBUF == 0

    @functools.partial(
        pl.kernel,
        out_type=jax.ShapeDtypeStruct((NC, N_PAD, D), jnp.float32),
        mesh=_mesh,
        scratch_types=[
            pltpu.VMEM((SB, CHUNK), jnp.int32),
            pltpu.VMEM((SB, CHUNK), jnp.int32),
        ] + [pltpu.VMEM((CHUNK, D), jnp.float32)] * NBUF + [
            pltpu.VMEM_SHARED((N_PAD, D), jnp.float32),
        ] + [pltpu.SemaphoreType.DMA] * NBUF,
    )
    def agg_kernel(h_hbm, edges_hbm, zrows_hbm, out_hbm,
                   src_v, dst_v, *rest):
        rows = rest[:NBUF]
        agg_sh = rest[NBUF]
        gsems = rest[NBUF + 1:]
        cid = lax.axis_index("c")
        sid = lax.axis_index("s")
        wid = sid * NC + cid
        n_me = jnp.where(wid == NW - 1, last_groups, groups)
        n_blocks = (n_me + SB - 1) // SB
        pltpu.sync_copy(zrows_hbm, agg_sh.at[pl.ds(sid * RPT, RPT)])
        plsc.subcore_barrier()

        def block(b, carry):
            pltpu.sync_copy(edges_hbm.at[0, wid, pl.ds(b * SB, SB)], src_v)
            pltpu.sync_copy(edges_hbm.at[1, wid, pl.ds(b * SB, SB)], dst_v)
            nblk = jnp.minimum(n_me - b * SB, SB)

            for k in range(NBUF):
                pltpu.async_copy(h_hbm.at[src_v.at[k]], rows[k], gsems[k])

            def body(i, carry2):
                for k in range(NBUF):
                    j = NBUF * i + k
                    pltpu.make_async_copy(h_hbm.at[src_v.at[j]], rows[k],
                                          gsems[k]).wait()
                    pltpu.sync_copy(rows[k], agg_sh.at[dst_v.at[j]],
                                    add=True)

                    @pl.when(j + NBUF < nblk)
                    def _():
                        pltpu.async_copy(h_hbm.at[src_v.at[j + NBUF]],
                                         rows[k], gsems[k])

                return carry2

            lax.fori_loop(0, nblk // NBUF, body, 0)
            return carry

        lax.fori_loop(0, n_blocks, block, 0)
        plsc.subcore_barrier()
        sl = pl.ds(sid * RPT, RPT)
        pltpu.sync_copy(agg_sh.at[sl], out_hbm.at[cid, sl])

    return agg_kernel


def _scale_body(feat_ref, hist_ref, h_ref):
    deg = hist_ref[0] + hist_ref[1]
    scale = lax.rsqrt(jnp.clip(deg, 1.0, None))
    h_ref[...] = feat_ref[...] * scale


def _final_body(p_ref, f0_ref, hist_ref, w_ref, b_ref, out_ref):
    agg = p_ref[0, :N, :] + p_ref[1, :N, :]
    deg = hist_ref[0] + hist_ref[1]
    scale = lax.rsqrt(jnp.clip(deg, 1.0, None))
    rst = agg * scale * (1.0 - ALPHA) + f0_ref[...] * ALPHA
    out_ref[...] = ((1.0 - BETA) * rst
                    + BETA * jnp.dot(rst, w_ref[...],
                                     preferred_element_type=jnp.float32)
                    + b_ref[...])


def kernel(feat, feat_0, edge_index, weight1, bias):
    num_edges = edge_index.shape[1]
    groups, last_groups = _split(num_edges)
    e_pad = NW * CHUNK * groups
    epw = groups * CHUNK
    assert epw % DCHUNK == 0
    dgroups = epw // DCHUNK
    dlast = (num_edges - (NW - 1) * epw) // DCHUNK
    edges_flat = jnp.pad(edge_index, ((0, 0), (0, e_pad - num_edges)))
    edges_deg = edges_flat.reshape(2, NW, dgroups, DCHUNK)
    edges = edges_flat.reshape(2, NW, groups, CHUNK)

    ones_c = jnp.ones((DCHUNK,), jnp.float32)
    zeros_1d = jnp.zeros((RPT,), jnp.float32)
    zeros_rows = jnp.zeros((RPT, D), jnp.float32)

    hists = _make_deg_kernel(dgroups, dlast)(edges_deg, ones_c, zeros_1d)
    hsrc = hists[:, 0, :N].reshape(NC, N, 1)
    hdst = hists[:, 1, :N].reshape(NC, N, 1)

    h = pl.pallas_call(
        _scale_body,
        out_shape=jax.ShapeDtypeStruct((N, D), jnp.float32),
    )(feat, hsrc)

    partials = _make_agg_kernel(groups, last_groups)(h, edges, zeros_rows)

    out = pl.pallas_call(
        _final_body,
        out_shape=jax.ShapeDtypeStruct((N, D), jnp.float32),
    )(partials, feat_0, hdst, weight1, bias.reshape(1, D))
    return out


# speedup vs baseline: 1.1111x; 1.1111x over previous
import functools
import math

import jax
import jax.numpy as jnp
from jax import lax
from jax.experimental import pallas as pl
from jax.experimental.pallas import tpu as pltpu
from jax.experimental.pallas import tpu_sc as plsc

N = 10000
D = 128
ALPHA = 0.1
LAMBDA = 1.0
LAYER = 4
BETA = math.log(LAMBDA / LAYER + 1.0)

NC = 2
NS = 16
NW = NC * NS
CHUNK = 64
NBUF = 4
DCHUNK = 128
SB = 32
N_PAD = 10240
RPT = N_PAD // NS

_mesh = plsc.VectorSubcoreMesh(core_axis_name="c", subcore_axis_name="s")


def _split(num_edges):
    assert num_edges % CHUNK == 0
    groups = -(-num_edges // (NW * CHUNK))
    groups += (-groups) % SB
    edges_per_worker = groups * CHUNK
    last_real = num_edges - (NW - 1) * edges_per_worker
    assert 0 < last_real <= edges_per_worker and last_real % CHUNK == 0
    assert last_real % NBUF == 0
    last_groups = last_real // CHUNK
    return groups, last_groups


def _make_deg_kernel(groups, last_groups):
    @functools.partial(
        pl.kernel,
        out_type=jax.ShapeDtypeStruct((NC, 2, N_PAD), jnp.float32),
        mesh=_mesh,
        scratch_types=[
            pltpu.VMEM((groups, DCHUNK), jnp.int32),
            pltpu.VMEM((groups, DCHUNK), jnp.int32),
            pltpu.VMEM((DCHUNK,), jnp.float32),
            pltpu.VMEM_SHARED((N_PAD,), jnp.float32),
            pltpu.VMEM_SHARED((N_PAD,), jnp.float32),
            pltpu.SemaphoreType.DMA,
        ],
    )
    def deg_kernel(edges_hbm, ones_hbm, zeros_hbm, out_hbm,
                   src_v, dst_v, ones_v, hsrc_sh, hdst_sh, sem):
        cid = lax.axis_index("c")
        sid = lax.axis_index("s")
        wid = sid * NC + cid
        n_me = jnp.where(wid == NW - 1, last_groups, groups)
        pltpu.sync_copy(edges_hbm.at[0, wid], src_v)
        pltpu.sync_copy(edges_hbm.at[1, wid], dst_v)
        pltpu.sync_copy(ones_hbm, ones_v)
        pltpu.sync_copy(zeros_hbm, hsrc_sh.at[pl.ds(sid * RPT, RPT)])
        pltpu.sync_copy(zeros_hbm, hdst_sh.at[pl.ds(sid * RPT, RPT)])
        plsc.subcore_barrier()

        def body(j, carry):
            d1 = pltpu.async_copy(ones_v, hsrc_sh.at[src_v.at[j]], sem,
                                  add=True)
            d2 = pltpu.async_copy(ones_v, hdst_sh.at[dst_v.at[j]], sem,
                                  add=True)
            d1.wait()
            d2.wait()
            return carry

        lax.fori_loop(0, n_me, body, 0)
        plsc.subcore_barrier()
        sl = pl.ds(sid * RPT, RPT)
        pltpu.sync_copy(hsrc_sh.at[sl], out_hbm.at[cid, 0, sl])
        pltpu.sync_copy(hdst_sh.at[sl], out_hbm.at[cid, 1, sl])

    return deg_kernel


def _make_agg_kernel(groups, last_groups):
    assert groups % SB == 0 and SB % NBUF == 0

    @functools.partial(
        pl.kernel,
        out_type=jax.ShapeDtypeStruct((NC, N_PAD, D), jnp.float32),
        mesh=_mesh,
        scratch_types=[
            pltpu.VMEM((SB, CHUNK), jnp.int32),
            pltpu.VMEM((SB, CHUNK), jnp.int32),
        ] + [pltpu.VMEM((CHUNK, D), jnp.float32)] * NBUF + [
            pltpu.VMEM_SHARED((N_PAD, D), jnp.float32),
        ] + [pltpu.SemaphoreType.DMA] * NBUF,
    )
    def agg_kernel(h_hbm, edges_hbm, zrows_hbm, out_hbm,
                   src_v, dst_v, *rest):
        rows = rest[:NBUF]
        agg_sh = rest[NBUF]
        gsems = rest[NBUF + 1:]
        cid = lax.axis_index("c")
        sid = lax.axis_index("s")
        wid = sid * NC + cid
        n_me = jnp.where(wid == NW - 1, last_groups, groups)
        n_blocks = (n_me + SB - 1) // SB
        pltpu.sync_copy(zrows_hbm, agg_sh.at[pl.ds(sid * RPT, RPT)])
        plsc.subcore_barrier()

        def block(b, carry):
            pltpu.sync_copy(edges_hbm.at[0, wid, pl.ds(b * SB, SB)], src_v)
            pltpu.sync_copy(edges_hbm.at[1, wid, pl.ds(b * SB, SB)], dst_v)
            nblk = jnp.minimum(n_me - b * SB, SB)

            for k in range(NBUF):
                pltpu.async_copy(h_hbm.at[src_v.at[k]], rows[k], gsems[k])

            def body(i, carry2):
                for k in range(NBUF):
                    j = NBUF * i + k
                    pltpu.make_async_copy(h_hbm.at[src_v.at[j]], rows[k],
                                          gsems[k]).wait()
                    pltpu.sync_copy(rows[k], agg_sh.at[dst_v.at[j]],
                                    add=True)

                    @pl.when(j + NBUF < nblk)
                    def _():
                        pltpu.async_copy(h_hbm.at[src_v.at[j + NBUF]],
                                         rows[k], gsems[k])

                return carry2

            lax.fori_loop(0, nblk // NBUF, body, 0)
            return carry

        lax.fori_loop(0, n_blocks, block, 0)
        plsc.subcore_barrier()
        sl = pl.ds(sid * RPT, RPT)
        pltpu.sync_copy(agg_sh.at[sl], out_hbm.at[cid, sl])

    return agg_kernel


def _scale_body(feat_ref, hist_ref, h_ref):
    deg = hist_ref[0] + hist_ref[1]
    scale = lax.rsqrt(jnp.clip(deg, 1.0, None))
    h_ref[...] = feat_ref[...] * scale


def _final_body(p_ref, f0_ref, hist_ref, w_ref, b_ref, out_ref):
    agg = p_ref[0, :N, :] + p_ref[1, :N, :]
    deg = hist_ref[0] + hist_ref[1]
    scale = lax.rsqrt(jnp.clip(deg, 1.0, None))
    rst = agg * scale * (1.0 - ALPHA) + f0_ref[...] * ALPHA
    out_ref[...] = ((1.0 - BETA) * rst
                    + BETA * jnp.dot(rst, w_ref[...],
                                     preferred_element_type=jnp.float32)
                    + b_ref[...])


def kernel(feat, feat_0, edge_index, weight1, bias):
    num_edges = edge_index.shape[1]
    groups, last_groups = _split(num_edges)
    e_pad = NW * CHUNK * groups
    epw = groups * CHUNK
    assert epw % DCHUNK == 0
    dgroups = epw // DCHUNK
    dlast = (num_edges - (NW - 1) * epw) // DCHUNK
    edges_flat = jnp.pad(edge_index, ((0, 0), (0, e_pad - num_edges)))
    edges_deg = edges_flat.reshape(2, NW, dgroups, DCHUNK)
    edges = edges_flat.reshape(2, NW, groups, CHUNK)

    ones_c = jnp.ones((DCHUNK,), jnp.float32)
    zeros_1d = jnp.zeros((RPT,), jnp.float32)
    zeros_rows = jnp.zeros((RPT, D), jnp.float32)

    hists = _make_deg_kernel(dgroups, dlast)(edges_deg, ones_c, zeros_1d)
    hsrc = hists[:, 0, :N].reshape(NC, N, 1)
    hdst = hists[:, 1, :N].reshape(NC, N, 1)

    h = pl.pallas_call(
        _scale_body,
        out_shape=jax.ShapeDtypeStruct((N, D), jnp.float32),
    )(feat, hsrc)

    partials = _make_agg_kernel(groups, last_groups)(h, edges, zeros_rows)

    out = pl.pallas_call(
        _final_body,
        out_shape=jax.ShapeDtypeStruct((N, D), jnp.float32),
    )(partials, feat_0, hdst, weight1, bias.reshape(1, D))
    return out


# speedup vs baseline: 1.1268x; 1.0142x over previous
import functools
import math

import jax
import jax.numpy as jnp
from jax import lax
from jax.experimental import pallas as pl
from jax.experimental.pallas import tpu as pltpu
from jax.experimental.pallas import tpu_sc as plsc

N = 10000
D = 128
ALPHA = 0.1
LAMBDA = 1.0
LAYER = 4
BETA = math.log(LAMBDA / LAYER + 1.0)

NC = 2
NS = 16
NW = NC * NS
CHUNK = 64
NBUF = 4
DCHUNK = 128
SB = 32
N_PAD = 10240
RPT = N_PAD // NS

_mesh = plsc.VectorSubcoreMesh(core_axis_name="c", subcore_axis_name="s")


def _split(num_edges):
    assert num_edges % CHUNK == 0
    groups = -(-num_edges // (NW * CHUNK))
    groups += (-groups) % SB
    edges_per_worker = groups * CHUNK
    last_real = num_edges - (NW - 1) * edges_per_worker
    assert 0 < last_real <= edges_per_worker and last_real % CHUNK == 0
    assert last_real % NBUF == 0
    last_groups = last_real // CHUNK
    return groups, last_groups


def _make_deg_kernel(groups, last_groups):
    @functools.partial(
        pl.kernel,
        out_type=jax.ShapeDtypeStruct((NC, 2, N_PAD), jnp.float32),
        mesh=_mesh,
        scratch_types=[
            pltpu.VMEM((groups, DCHUNK), jnp.int32),
            pltpu.VMEM((groups, DCHUNK), jnp.int32),
            pltpu.VMEM((DCHUNK,), jnp.float32),
            pltpu.VMEM_SHARED((N_PAD,), jnp.float32),
            pltpu.VMEM_SHARED((N_PAD,), jnp.float32),
            pltpu.SemaphoreType.DMA,
        ],
    )
    def deg_kernel(edges_hbm, ones_hbm, zeros_hbm, out_hbm,
                   src_v, dst_v, ones_v, hsrc_sh, hdst_sh, sem):
        cid = lax.axis_index("c")
        sid = lax.axis_index("s")
        wid = sid * NC + cid
        n_me = jnp.where(wid == NW - 1, last_groups, groups)
        pltpu.sync_copy(edges_hbm.at[0, wid], src_v)
        pltpu.sync_copy(edges_hbm.at[1, wid], dst_v)
        pltpu.sync_copy(ones_hbm, ones_v)
        pltpu.sync_copy(zeros_hbm, hsrc_sh.at[pl.ds(sid * RPT, RPT)])
        pltpu.sync_copy(zeros_hbm, hdst_sh.at[pl.ds(sid * RPT, RPT)])
        plsc.subcore_barrier()

        def body(i, carry):
            j0 = 2 * i
            j1 = j0 + 1
            ds = [
                pltpu.async_copy(ones_v, hsrc_sh.at[src_v.at[j0]], sem,
                                 add=True),
                pltpu.async_copy(ones_v, hdst_sh.at[dst_v.at[j0]], sem,
                                 add=True),
                pltpu.async_copy(ones_v, hsrc_sh.at[src_v.at[j1]], sem,
                                 add=True),
                pltpu.async_copy(ones_v, hdst_sh.at[dst_v.at[j1]], sem,
                                 add=True),
            ]
            for d in ds:
                d.wait()
            return carry

        lax.fori_loop(0, n_me // 2, body, 0)
        plsc.subcore_barrier()
        sl = pl.ds(sid * RPT, RPT)
        pltpu.sync_copy(hsrc_sh.at[sl], out_hbm.at[cid, 0, sl])
        pltpu.sync_copy(hdst_sh.at[sl], out_hbm.at[cid, 1, sl])

    return deg_kernel


def _make_agg_kernel(groups, last_groups):
    assert groups % SB == 0 and SB % NBUF == 0

    @functools.partial(
        pl.kernel,
        out_type=jax.ShapeDtypeStruct((NC, N_PAD, D), jnp.float32),
        mesh=_mesh,
        scratch_types=[
            pltpu.VMEM((SB, CHUNK), jnp.int32),
            pltpu.VMEM((SB, CHUNK), jnp.int32),
        ] + [pltpu.VMEM((CHUNK, D), jnp.float32)] * NBUF + [
            pltpu.VMEM_SHARED((N_PAD, D), jnp.float32),
        ] + [pltpu.SemaphoreType.DMA] * NBUF,
    )
    def agg_kernel(h_hbm, edges_hbm, zrows_hbm, out_hbm,
                   src_v, dst_v, *rest):
        rows = rest[:NBUF]
        agg_sh = rest[NBUF]
        gsems = rest[NBUF + 1:]
        cid = lax.axis_index("c")
        sid = lax.axis_index("s")
        wid = sid * NC + cid
        n_me = jnp.where(wid == NW - 1, last_groups, groups)
        n_blocks = (n_me + SB - 1) // SB
        pltpu.sync_copy(zrows_hbm, agg_sh.at[pl.ds(sid * RPT, RPT)])
        plsc.subcore_barrier()

        def block(b, carry):
            pltpu.sync_copy(edges_hbm.at[0, wid, pl.ds(b * SB, SB)], src_v)
            pltpu.sync_copy(edges_hbm.at[1, wid, pl.ds(b * SB, SB)], dst_v)
            nblk = jnp.minimum(n_me - b * SB, SB)

            for k in range(NBUF):
                pltpu.async_copy(h_hbm.at[src_v.at[k]], rows[k], gsems[k])

            def body(i, carry2):
                for k in range(NBUF):
                    j = NBUF * i + k
                    pltpu.make_async_copy(h_hbm.at[src_v.at[j]], rows[k],
                                          gsems[k]).wait()
                    pltpu.sync_copy(rows[k], agg_sh.at[dst_v.at[j]],
                                    add=True)

                    @pl.when(j + NBUF < nblk)
                    def _():
                        pltpu.async_copy(h_hbm.at[src_v.at[j + NBUF]],
                                         rows[k], gsems[k])

                return carry2

            lax.fori_loop(0, nblk // NBUF, body, 0)
            return carry

        lax.fori_loop(0, n_blocks, block, 0)
        plsc.subcore_barrier()
        sl = pl.ds(sid * RPT, RPT)
        pltpu.sync_copy(agg_sh.at[sl], out_hbm.at[cid, sl])

    return agg_kernel


def _scale_body(feat_ref, hist_ref, h_ref):
    deg = hist_ref[0] + hist_ref[1]
    scale = lax.rsqrt(jnp.clip(deg, 1.0, None))
    h_ref[...] = feat_ref[...] * scale


def _final_body(p_ref, f0_ref, hist_ref, w_ref, b_ref, out_ref):
    agg = p_ref[0, :N, :] + p_ref[1, :N, :]
    deg = hist_ref[0] + hist_ref[1]
    scale = lax.rsqrt(jnp.clip(deg, 1.0, None))
    rst = agg * scale * (1.0 - ALPHA) + f0_ref[...] * ALPHA
    out_ref[...] = ((1.0 - BETA) * rst
                    + BETA * jnp.dot(rst, w_ref[...],
                                     preferred_element_type=jnp.float32)
                    + b_ref[...])


def kernel(feat, feat_0, edge_index, weight1, bias):
    num_edges = edge_index.shape[1]
    groups, last_groups = _split(num_edges)
    e_pad = NW * CHUNK * groups
    epw = groups * CHUNK
    assert epw % DCHUNK == 0
    dgroups = epw // DCHUNK
    dlast = (num_edges - (NW - 1) * epw) // DCHUNK
    edges_flat = jnp.pad(edge_index, ((0, 0), (0, e_pad - num_edges)))
    edges_deg = edges_flat.reshape(2, NW, dgroups, DCHUNK)
    edges = edges_flat.reshape(2, NW, groups, CHUNK)

    ones_c = jnp.ones((DCHUNK,), jnp.float32)
    zeros_1d = jnp.zeros((RPT,), jnp.float32)
    zeros_rows = jnp.zeros((RPT, D), jnp.float32)

    hists = _make_deg_kernel(dgroups, dlast)(edges_deg, ones_c, zeros_1d)
    hsrc = hists[:, 0, :N].reshape(NC, N, 1)
    hdst = hists[:, 1, :N].reshape(NC, N, 1)

    h = pl.pallas_call(
        _scale_body,
        out_shape=jax.ShapeDtypeStruct((N, D), jnp.float32),
    )(feat, hsrc)

    partials = _make_agg_kernel(groups, last_groups)(h, edges, zeros_rows)

    out = pl.pallas_call(
        _final_body,
        out_shape=jax.ShapeDtypeStruct((N, D), jnp.float32),
    )(partials, feat_0, hdst, weight1, bias.reshape(1, D))
    return out
